# baseline (speedup 1.0000x reference)
import functools

import jax
import jax.numpy as jnp
import numpy as np
from jax import lax
from jax.experimental import pallas as pl
from jax.experimental.pallas import tpu as pltpu
from jax.experimental.pallas import tpu_sc as plsc

_NUM_ROWS = 1000000
_EMBED = 16
_R_MASK = np.int32(65535)
_R_ROWS = 65536
_Q_ROWS = 32768

_NC = 2
_NS = 16
_NW = _NC * _NS
_L = 16

_CHUNK = 512
_NBUF = 2

_Q_PERIOD = 15625
_INV_QP = np.float32(1.0 / _Q_PERIOD)


@functools.partial(jax.jit, static_argnames=("batch", "fields"))
def _lookup_add(ids_flat, sub_table, batch, fields):
    n = batch * fields
    per_w = n // _NW
    n_chunks = per_w // _CHUNK
    vecs_per_chunk = _CHUNK // _L
    chunks_per_field = batch // _CHUNK
    assert per_w % _CHUNK == 0 and batch % _CHUNK == 0
    assert chunks_per_field & (chunks_per_field - 1) == 0
    cpf_shift = chunks_per_field.bit_length() - 1

    mesh = plsc.VectorSubcoreMesh(core_axis_name="c", subcore_axis_name="s")

    @functools.partial(
        pl.kernel,
        out_type=jax.ShapeDtypeStruct((fields, _EMBED, batch), jnp.float32),
        mesh=mesh,
        scratch_types=[
            pltpu.VMEM((per_w,), jnp.int32),
            pltpu.VMEM((per_w,), jnp.int32),
            [pltpu.VMEM((_CHUNK, _EMBED), jnp.float32) for _ in range(_NBUF)],
            [pltpu.VMEM((_CHUNK, _EMBED), jnp.float32) for _ in range(_NBUF)],
            [pltpu.VMEM((_EMBED, _CHUNK), jnp.float32) for _ in range(_NBUF)],
            [pltpu.SemaphoreType.DMA for _ in range(_NBUF)],
            [pltpu.SemaphoreType.DMA for _ in range(_NBUF)],
        ],
        compiler_params=pltpu.CompilerParams(
            use_tc_tiling_on_sc=False, needs_layout_passes=False),
    )
    def sc_kernel(ids_hbm, sub_hbm, out_hbm,
                  idxq_v, idxr_v, bq, br, bo, sg, sw):
        wid = lax.axis_index("s") * _NC + lax.axis_index("c")
        base = wid * per_w

        pltpu.sync_copy(ids_hbm.at[pl.ds(base, per_w)], idxq_v)

        def compute_idx(cc):
            def body(j, carry):
                o = cc * _CHUNK + j * _L
                x = idxq_v[pl.ds(o, _L)]
                idxr_v[pl.ds(o, _L)] = x & _R_MASK
                t = lax.shift_right_logical(x, 6) & np.int32(0x1FFFC00)
                q = (t.astype(jnp.float32) * _INV_QP).astype(jnp.int32)
                r = t - q * _Q_PERIOD
                r = jnp.where(r < 0, r + _Q_PERIOD, r)
                r = jnp.where(r >= _Q_PERIOD, r - _Q_PERIOD, r)
                idxq_v[pl.ds(o, _L)] = r + _R_ROWS
                return carry
            lax.fori_loop(0, vecs_per_chunk, body, 0, unroll=4)

        def issue(cc):
            p = cc % _NBUF
            co = cc * _CHUNK
            dq = pltpu.async_copy(
                sub_hbm.at[idxq_v.at[pl.ds(co, _CHUNK)]], bq[p], sg[p])
            dr = pltpu.async_copy(
                sub_hbm.at[idxr_v.at[pl.ds(co, _CHUNK)]], br[p], sg[p])
            return dq, dr

        lane = lax.iota(jnp.int32, _L)
        col0 = lane * 0

        gth = {}
        for cc in range(_NBUF):
            compute_idx(cc)
            gth[cc] = issue(cc)

        wb = {}
        for cc in range(n_chunks):
            p = cc % _NBUF
            nxt = cc + _NBUF
            if nxt < n_chunks:
                compute_idx(nxt)
            dq, dr = gth.pop(cc)
            dq.wait()
            dr.wait()
            if cc >= _NBUF:
                wb.pop(cc - _NBUF).wait()

            def add_body(i, col, p=p):
                v = bq[p][i] + br[p][i]
                plsc.store_scatter(bo[p], (lane, col), v)
                return col + 1
            lax.fori_loop(0, _CHUNK, add_body, col0, unroll=8)

            g = wid * n_chunks + cc
            f = lax.shift_right_logical(g, cpf_shift)
            b0 = (g & (chunks_per_field - 1)) * _CHUNK
            wb[cc] = pltpu.async_copy(
                bo[p], out_hbm.at[f].at[:, pl.ds(b0, _CHUNK)], sw[p])
            if nxt < n_chunks:
                gth[nxt] = issue(nxt)

        for cc in sorted(wb):
            wb[cc].wait()

    return sc_kernel(ids_flat, sub_table)


def kernel(ids, table):
    batch, fields = ids.shape
    sub_table = jnp.concatenate([table[:_R_ROWS], table[::64]], axis=0)
    out = _lookup_add(ids.T.reshape(-1), sub_table, batch, fields)
    return jnp.transpose(out, (2, 0, 1))


# speedup vs baseline: 1.0981x; 1.0981x over previous
import functools

import jax
import jax.numpy as jnp
import numpy as np
from jax import lax
from jax.experimental import pallas as pl
from jax.experimental.pallas import tpu as pltpu
from jax.experimental.pallas import tpu_sc as plsc

_NUM_ROWS = 1000000
_EMBED = 16
_R_MASK = np.int32(65535)
_R_ROWS = 65536
_Q_ROWS = 32768

_NC = 2
_NS = 16
_NW = _NC * _NS
_L = 16

_CHUNK = 512
_NBUF = 2

_Q_PERIOD = 15625
_INV_QP = np.float32(1.0 / _Q_PERIOD)


@functools.partial(jax.jit, static_argnames=("batch", "fields"))
def _lookup_add(ids_flat, sub_table, batch, fields):
    n = batch * fields
    per_w = n // _NW
    n_chunks = per_w // _CHUNK
    vecs_per_chunk = _CHUNK // _L
    chunks_per_field = batch // _CHUNK
    assert per_w % _CHUNK == 0 and batch % _CHUNK == 0
    assert chunks_per_field & (chunks_per_field - 1) == 0
    cpf_shift = chunks_per_field.bit_length() - 1

    mesh = plsc.VectorSubcoreMesh(core_axis_name="c", subcore_axis_name="s")

    @functools.partial(
        pl.kernel,
        out_type=jax.ShapeDtypeStruct((fields, _EMBED, batch), jnp.float32),
        mesh=mesh,
        scratch_types=[
            pltpu.VMEM((per_w,), jnp.int32),
            pltpu.VMEM((per_w,), jnp.int32),
            [pltpu.VMEM((_CHUNK, _EMBED), jnp.float32) for _ in range(_NBUF)],
            [pltpu.VMEM((_CHUNK, _EMBED), jnp.float32) for _ in range(_NBUF)],
            [pltpu.VMEM((_EMBED, _CHUNK), jnp.float32) for _ in range(_NBUF)],
            [pltpu.SemaphoreType.DMA for _ in range(_NBUF)],
            [pltpu.SemaphoreType.DMA for _ in range(_NBUF)],
        ],
        compiler_params=pltpu.CompilerParams(
            use_tc_tiling_on_sc=False, needs_layout_passes=False),
    )
    def sc_kernel(ids_hbm, sub_hbm, out_hbm,
                  idxq_v, idxr_v, bq, br, bo, sg, sw):
        wid = lax.axis_index("s") * _NC + lax.axis_index("c")
        base = wid * per_w

        pltpu.sync_copy(ids_hbm.at[pl.ds(base, per_w)], idxq_v)

        def compute_idx(cc):
            def body(j, carry):
                o = cc * _CHUNK + j * _L
                x = idxq_v[pl.ds(o, _L)]
                idxr_v[pl.ds(o, _L)] = x & _R_MASK
                t = lax.shift_right_logical(x, 6) & np.int32(0x1FFFC00)
                q = (t.astype(jnp.float32) * _INV_QP).astype(jnp.int32)
                r = t - q * _Q_PERIOD
                r = jnp.where(r < 0, r + _Q_PERIOD, r)
                r = jnp.where(r >= _Q_PERIOD, r - _Q_PERIOD, r)
                idxq_v[pl.ds(o, _L)] = r + _R_ROWS
                return carry
            lax.fori_loop(0, vecs_per_chunk, body, 0, unroll=4)

        def issue(cc):
            p = cc % _NBUF
            co = cc * _CHUNK
            dq = pltpu.async_copy(
                sub_hbm.at[idxq_v.at[pl.ds(co, _CHUNK)]], bq[p], sg[p])
            dr = pltpu.async_copy(
                sub_hbm.at[idxr_v.at[pl.ds(co, _CHUNK)]], br[p], sg[p])
            return dq, dr

        lane = lax.iota(jnp.int32, _L)
        col0 = lane * 0

        gth = {}
        for cc in range(_NBUF):
            compute_idx(cc)
            gth[cc] = issue(cc)

        wb = {}
        for cc in range(n_chunks):
            p = cc % _NBUF
            nxt = cc + _NBUF
            if nxt < n_chunks:
                compute_idx(nxt)
            dq, dr = gth.pop(cc)
            dq.wait()
            dr.wait()
            if cc >= _NBUF:
                wb.pop(cc - _NBUF).wait()

            def add_body(i, col, p=p):
                v = bq[p][i] + br[p][i]
                plsc.store_scatter(bo[p], (lane, col), v)
                return col + 1
            lax.fori_loop(0, _CHUNK, add_body, col0, unroll=8)

            g = wid * n_chunks + cc
            f = lax.shift_right_logical(g, cpf_shift)
            b0 = (g & (chunks_per_field - 1)) * _CHUNK
            wb[cc] = pltpu.async_copy(
                bo[p], out_hbm.at[f].at[:, pl.ds(b0, _CHUNK)], sw[p])
            if nxt < n_chunks:
                gth[nxt] = issue(nxt)

        for cc in sorted(wb):
            wb[cc].wait()

    return sc_kernel(ids_flat, sub_table)


def kernel(ids, table):
    batch, fields = ids.shape
    qrows = jnp.arange(_Q_PERIOD, dtype=jnp.int32) * np.int32(64)
    sub_table = jnp.concatenate([table[:_R_ROWS], table[qrows]], axis=0)
    out = _lookup_add(ids.T.reshape(-1), sub_table, batch, fields)
    return jnp.transpose(out, (2, 0, 1))
